# baseline (speedup 1.0000x reference)
import jax
import jax.numpy as jnp
from jax import lax
from jax.experimental import pallas as pl
from jax.experimental.pallas import tpu as pltpu
from jax.experimental.pallas import tpu_sc as plsc

N = 10000
NPAD = 10240
D = 256
H = 128
E = 160000
NC = 2
NS = 16

EC = 80
ECHUNKS = E // EC
TCHUNKS = ECHUNKS // NS
NBUF = 4
ZR = 624

DC = 40
DCHUNKS = E // DC
DTCHUNKS = DCHUNKS // (NC * NS)
NPT = NPAD // NS

BM = 10000
GRID = N // BM


def _sc_mesh():
    return plsc.VectorSubcoreMesh(
        core_axis_name="c", subcore_axis_name="s",
        num_cores=NC, num_subcores=NS)


DNB = 4


def _deg_body(dst2_hbm, zeros_hbm, ones_hbm, deg_out,
              acc, didx, ones_v, s0, s1, s2, s3):
    sems = [s0, s1, s2, s3]
    c = lax.axis_index("c")
    s = lax.axis_index("s")
    w = c * NS + s
    nb = s * NPT

    pltpu.sync_copy(ones_hbm, ones_v)
    pltpu.sync_copy(zeros_hbm.at[pl.ds(nb, NPT)], acc.at[pl.ds(nb, NPT)])
    pltpu.sync_copy(dst2_hbm.at[w], didx)
    plsc.subcore_barrier()

    for j in range(DNB):
        pltpu.async_copy(ones_v, acc.at[didx.at[j]], sems[j], add=True)

    def _body(i, carry):
        for j in range(DNB):
            k = DNB * i + j
            pltpu.make_async_copy(ones_v, acc.at[didx.at[0]], sems[j]).wait()
            kn = k + DNB

            @pl.when(kn < DTCHUNKS)
            def _():
                pltpu.async_copy(ones_v, acc.at[didx.at[kn]], sems[j], add=True)
        return carry
    lax.fori_loop(0, (DTCHUNKS - 1) // DNB, _body, 0)
    pltpu.make_async_copy(ones_v, acc.at[didx.at[0]], sems[0]).wait()

    plsc.subcore_barrier()
    pltpu.sync_copy(acc.at[pl.ds(nb, NPT)], deg_out.at[c].at[pl.ds(nb, NPT)])


def _make_deg_kernel():
    return pl.kernel(
        _deg_body,
        out_type=jax.ShapeDtypeStruct((2, NPAD), jnp.float32),
        mesh=_sc_mesh(),
        scratch_types=[
            pltpu.VMEM_SHARED((NPAD,), jnp.float32),
            pltpu.VMEM((DTCHUNKS, DC), jnp.int32),
            pltpu.VMEM((DC,), jnp.float32),
            pltpu.SemaphoreType.DMA,
            pltpu.SemaphoreType.DMA,
            pltpu.SemaphoreType.DMA,
            pltpu.SemaphoreType.DMA,
        ],
    )


def _edge_body(g, src, dst3, agg,
               acc, si0, si1, si2, si3, di0, di1, di2, di3,
               r0, r1, r2, r3, gs0, gs1, gs2, gs3, is0, is1, is2, is3,
               ss0, ss1, ss2, ss3, dc0, dc1, dc2, dc3):
    sidx = [si0, si1, si2, si3]
    didx = [di0, di1, di2, di3]
    rows = [r0, r1, r2, r3]
    gsems = [gs0, gs1, gs2, gs3]
    isems = [is0, is1, is2, is3]
    ssems = [ss0, ss1, ss2, ss3]
    dcopy = [dc0, dc1, dc2, dc3]
    c = lax.axis_index("c")
    s = lax.axis_index("s")
    rb = s * TCHUNKS
    goff = c * N

    def idx_load(j, kg):
        pltpu.async_copy(src.at[pl.ds((rb + kg) * EC, EC)], sidx[j], isems[j])
        pltpu.async_copy(dst3.at[rb + kg], didx[j], isems[j])

    def idx_wait_adj(j):
        pltpu.make_async_copy(src.at[pl.ds(0, EC)], sidx[j], isems[j]).wait()
        pltpu.make_async_copy(dst3.at[0], didx[j], isems[j]).wait()
        for l in range(EC // 16):
            sidx[j][pl.ds(l * 16, 16)] = sidx[j][pl.ds(l * 16, 16)] + goff

    def gstart(j):
        pltpu.async_copy(g.at[sidx[j]], rows[j], gsems[j])

    def gwait(j):
        pltpu.make_async_copy(g.at[sidx[j]], rows[j], gsems[j]).wait()

    def sstart(j):
        dv = didx[j].at[0]
        cv = dcopy[j].at[0]
        for l in range(EC // 16):
            cv[pl.ds(l * 16, 16)] = dv[pl.ds(l * 16, 16)]
        pltpu.async_copy(rows[j], acc.at[dcopy[j].at[0]], ssems[j], add=True)

    def swait(j):
        pltpu.make_async_copy(rows[j], acc.at[dcopy[j].at[0]], ssems[j]).wait()

    for j in range(NBUF):
        idx_load(j, j)

    def _zf(i, carry):
        for j in range(H // 16):
            r3[i, pl.ds(j * 16, 16)] = jnp.zeros((16,), jnp.float32)
        return carry
    lax.fori_loop(0, EC, _zf, 0)
    for j in range(3):
        idx_wait_adj(j)
        gstart(j)

    @pl.when(s < 15)
    def _():
        for t in range(7):
            pltpu.sync_copy(r3, acc.at[pl.ds(s * ZR + t * EC, EC)])
        pltpu.sync_copy(r3.at[pl.ds(0, ZR - 7 * EC)],
                        acc.at[pl.ds(s * ZR + 7 * EC, ZR - 7 * EC)])

    @pl.when(s == 15)
    def _():
        for t in range(8):
            pltpu.sync_copy(r3, acc.at[pl.ds(15 * ZR + t * EC, EC)])

    plsc.subcore_barrier()

    def _body(i, carry):
        for j in range(NBUF):
            k = NBUF * i + j
            gwait(j)
            sstart(j)
            kn4 = k + NBUF

            @pl.when(kn4 < TCHUNKS)
            def _():
                idx_load(j, kn4)
            kn3 = k + 3
            j3 = (j + 3) % NBUF

            @pl.when(kn3 < TCHUNKS)
            def _():
                @pl.when(kn3 >= NBUF)
                def _():
                    swait(j3)
                idx_wait_adj(j3)
                gstart(j3)
        return carry
    lax.fori_loop(0, (TCHUNKS - 1) // NBUF, _body, 0)
    gwait(0)
    sstart(0)
    swait(1)
    swait(2)
    swait(3)
    swait(0)

    plsc.subcore_barrier()

    @pl.when(s < 15)
    def _():
        pltpu.sync_copy(acc.at[pl.ds(s * ZR, ZR)],
                        agg.at[c].at[pl.ds(s * ZR, ZR)])

    @pl.when(s == 15)
    def _():
        pltpu.sync_copy(acc.at[pl.ds(15 * ZR, N - 15 * ZR)],
                        agg.at[c].at[pl.ds(15 * ZR, N - 15 * ZR)])


def _make_edge_kernel():
    sem = pltpu.SemaphoreType.DMA
    return pl.kernel(
        _edge_body,
        out_type=jax.ShapeDtypeStruct((2, N, H), jnp.float32),
        mesh=_sc_mesh(),
        scratch_types=(
            [pltpu.VMEM_SHARED((N, H), jnp.float32)]
            + [pltpu.VMEM((EC,), jnp.int32) for _ in range(NBUF)]
            + [pltpu.VMEM((1, EC), jnp.int32) for _ in range(NBUF)]
            + [pltpu.VMEM((EC, H), jnp.float32) for _ in range(NBUF)]
            + [sem] * (3 * NBUF)
            + [pltpu.VMEM((1, EC), jnp.int32) for _ in range(NBUF)]

        ),
    )


def _mm0_body(deg_ref, x_ref, w_ref, h2_ref, g_ref, dis_ref):
    deg = deg_ref[0] + deg_ref[1] + 1.0
    dis = lax.rsqrt(deg)
    h2 = jnp.dot(x_ref[...], w_ref[...], preferred_element_type=jnp.float32)
    g = h2 * dis
    h2_ref[...] = h2.astype(jnp.bfloat16)
    g_ref[...] = jnp.stack([g[:, :H], g[:, H:]])
    dis_ref[...] = dis


def _mm_body(agg_ref, h2p_ref, dis_ref, b_ref, w_ref, h2_ref, g_ref):
    dis = dis_ref[...]
    agg = jnp.concatenate([agg_ref[0], agg_ref[1]], axis=-1)
    h2p = h2p_ref[...].astype(jnp.float32)
    hin = dis * agg + h2p * (dis * dis) + b_ref[...]
    hin = jnp.maximum(hin, 0.0)
    h2 = jnp.dot(hin, w_ref[...], preferred_element_type=jnp.float32)
    g = h2 * dis
    h2_ref[...] = h2.astype(jnp.bfloat16)
    g_ref[...] = jnp.stack([g[:, :H], g[:, H:]])


def _epi_body(agg_ref, h2p_ref, dis_ref, b_ref, out_ref):
    dis = dis_ref[...]
    agg = jnp.concatenate([agg_ref[0], agg_ref[1]], axis=-1)
    h2p = h2p_ref[...].astype(jnp.float32)
    out_ref[...] = dis * agg + h2p * (dis * dis) + b_ref[...]


def _row_spec(shape_tail):
    return pl.BlockSpec((BM,) + shape_tail,
                        lambda i: (i,) + (0,) * len(shape_tail))


def _pair_spec():
    return pl.BlockSpec((2, BM, H), lambda i: (0, i, 0))


def _make_mm0():
    return pl.pallas_call(
        _mm0_body,
        grid=(GRID,),
        in_specs=[
            pl.BlockSpec((2, BM, 1), lambda i: (0, i, 0)),
            _row_spec((D,)),
            pl.BlockSpec((D, D), lambda i: (0, 0)),
        ],
        out_specs=[_row_spec((D,)), _pair_spec(), _row_spec((1,))],
        out_shape=[jax.ShapeDtypeStruct((N, D), jnp.bfloat16),
                   jax.ShapeDtypeStruct((2, N, H), jnp.float32),
                   jax.ShapeDtypeStruct((N, 1), jnp.float32)],
    )


def _make_mm():
    return pl.pallas_call(
        _mm_body,
        grid=(GRID,),
        in_specs=[
            _pair_spec(), _row_spec((D,)), _row_spec((1,)),
            pl.BlockSpec((1, D), lambda i: (0, 0)),
            pl.BlockSpec((D, D), lambda i: (0, 0)),
        ],
        out_specs=[_row_spec((D,)), _pair_spec()],
        out_shape=[jax.ShapeDtypeStruct((N, D), jnp.bfloat16),
                   jax.ShapeDtypeStruct((2, N, H), jnp.float32)],
    )


def _make_epi():
    return pl.pallas_call(
        _epi_body,
        grid=(GRID,),
        in_specs=[
            _pair_spec(), _row_spec((D,)), _row_spec((1,)),
            pl.BlockSpec((1, D), lambda i: (0, 0)),
        ],
        out_specs=_row_spec((D,)),
        out_shape=jax.ShapeDtypeStruct((N, D), jnp.float32),
    )


def kernel(x, adj_t, W0, b0, W1, b1, W2, b2):
    src = adj_t[0]
    dst3 = adj_t[1].reshape(ECHUNKS, 1, EC)
    zeros1 = jnp.zeros((NPAD,), jnp.float32)
    ones1 = jnp.ones((DC,), jnp.float32)
    b0r = b0.reshape(1, D)
    b1r = b1.reshape(1, D)
    b2r = b2.reshape(1, D)

    deg_k = _make_deg_kernel()
    edge_k = _make_edge_kernel()
    mm0 = _make_mm0()
    mm = _make_mm()
    epi = _make_epi()

    deg1 = deg_k(adj_t[1].reshape(NC * NS, DTCHUNKS, DC), zeros1, ones1)
    h2, gp, dis = mm0(deg1.reshape(2, NPAD, 1)[:, :N], x, W0)
    aggp = edge_k(gp.reshape(2 * N, H), src, dst3)
    h2, gp = mm(aggp, h2, dis, b0r, W1)
    aggp = edge_k(gp.reshape(2 * N, H), src, dst3)
    h2, gp = mm(aggp, h2, dis, b1r, W2)
    aggp = edge_k(gp.reshape(2 * N, H), src, dst3)
    return epi(aggp, h2, dis, b2r)


# speedup vs baseline: 1.0407x; 1.0407x over previous
import jax
import jax.numpy as jnp
from jax import lax
from jax.experimental import pallas as pl
from jax.experimental.pallas import tpu as pltpu
from jax.experimental.pallas import tpu_sc as plsc

N = 10000
NPAD = 10240
D = 256
H = 128
E = 160000
NC = 2
NS = 16

EC = 80
ECHUNKS = E // EC
TCHUNKS = ECHUNKS // NS
NBUF = 4
ZR = 624

DC = 40
DCHUNKS = E // DC
DTCHUNKS = DCHUNKS // (NC * NS)
NPT = NPAD // NS

BM = 5000
GRID = N // BM


def _sc_mesh():
    return plsc.VectorSubcoreMesh(
        core_axis_name="c", subcore_axis_name="s",
        num_cores=NC, num_subcores=NS)


DNB = 4


def _deg_body(dst2_hbm, zeros_hbm, ones_hbm, deg_out,
              acc, didx, ones_v, s0, s1, s2, s3):
    sems = [s0, s1, s2, s3]
    c = lax.axis_index("c")
    s = lax.axis_index("s")
    w = c * NS + s
    nb = s * NPT

    pltpu.sync_copy(ones_hbm, ones_v)
    pltpu.sync_copy(zeros_hbm.at[pl.ds(nb, NPT)], acc.at[pl.ds(nb, NPT)])
    pltpu.sync_copy(dst2_hbm.at[w], didx)
    plsc.subcore_barrier()

    for j in range(DNB):
        pltpu.async_copy(ones_v, acc.at[didx.at[j]], sems[j], add=True)

    def _body(i, carry):
        for j in range(DNB):
            k = DNB * i + j
            pltpu.make_async_copy(ones_v, acc.at[didx.at[0]], sems[j]).wait()
            kn = k + DNB

            @pl.when(kn < DTCHUNKS)
            def _():
                pltpu.async_copy(ones_v, acc.at[didx.at[kn]], sems[j], add=True)
        return carry
    lax.fori_loop(0, (DTCHUNKS - 1) // DNB, _body, 0)
    pltpu.make_async_copy(ones_v, acc.at[didx.at[0]], sems[0]).wait()

    plsc.subcore_barrier()
    pltpu.sync_copy(acc.at[pl.ds(nb, NPT)], deg_out.at[c].at[pl.ds(nb, NPT)])


def _make_deg_kernel():
    return pl.kernel(
        _deg_body,
        out_type=jax.ShapeDtypeStruct((2, NPAD), jnp.float32),
        mesh=_sc_mesh(),
        scratch_types=[
            pltpu.VMEM_SHARED((NPAD,), jnp.float32),
            pltpu.VMEM((DTCHUNKS, DC), jnp.int32),
            pltpu.VMEM((DC,), jnp.float32),
            pltpu.SemaphoreType.DMA,
            pltpu.SemaphoreType.DMA,
            pltpu.SemaphoreType.DMA,
            pltpu.SemaphoreType.DMA,
        ],
    )


def _edge_body(g, src, dst3, agg,
               acc, si0, si1, si2, si3, di0, di1, di2, di3,
               r0, r1, r2, r3, gs0, gs1, gs2, gs3, is0, is1, is2, is3,
               ss0, ss1, ss2, ss3, dc0, dc1, dc2, dc3):
    sidx = [si0, si1, si2, si3]
    didx = [di0, di1, di2, di3]
    rows = [r0, r1, r2, r3]
    gsems = [gs0, gs1, gs2, gs3]
    isems = [is0, is1, is2, is3]
    ssems = [ss0, ss1, ss2, ss3]
    dcopy = [dc0, dc1, dc2, dc3]
    c = lax.axis_index("c")
    s = lax.axis_index("s")
    rb = s * TCHUNKS
    goff = c * N

    def idx_load(j, kg):
        pltpu.async_copy(src.at[pl.ds((rb + kg) * EC, EC)], sidx[j], isems[j])
        pltpu.async_copy(dst3.at[rb + kg], didx[j], isems[j])

    def idx_wait_adj(j):
        pltpu.make_async_copy(src.at[pl.ds(0, EC)], sidx[j], isems[j]).wait()
        pltpu.make_async_copy(dst3.at[0], didx[j], isems[j]).wait()
        for l in range(EC // 16):
            sidx[j][pl.ds(l * 16, 16)] = sidx[j][pl.ds(l * 16, 16)] + goff

    def gstart(j):
        pltpu.async_copy(g.at[sidx[j]], rows[j], gsems[j])

    def gwait(j):
        pltpu.make_async_copy(g.at[sidx[j]], rows[j], gsems[j]).wait()

    def sstart(j):
        dv = didx[j].at[0]
        cv = dcopy[j].at[0]
        for l in range(EC // 16):
            cv[pl.ds(l * 16, 16)] = dv[pl.ds(l * 16, 16)]
        pltpu.async_copy(rows[j], acc.at[dcopy[j].at[0]], ssems[j], add=True)

    def swait(j):
        pltpu.make_async_copy(rows[j], acc.at[dcopy[j].at[0]], ssems[j]).wait()

    for j in range(NBUF):
        idx_load(j, j)

    def _zf(i, carry):
        for j in range(H // 16):
            r3[i, pl.ds(j * 16, 16)] = jnp.zeros((16,), jnp.float32)
        return carry
    lax.fori_loop(0, EC, _zf, 0)
    for j in range(3):
        idx_wait_adj(j)
        gstart(j)

    @pl.when(s < 15)
    def _():
        for t in range(7):
            pltpu.sync_copy(r3, acc.at[pl.ds(s * ZR + t * EC, EC)])
        pltpu.sync_copy(r3.at[pl.ds(0, ZR - 7 * EC)],
                        acc.at[pl.ds(s * ZR + 7 * EC, ZR - 7 * EC)])

    @pl.when(s == 15)
    def _():
        for t in range(8):
            pltpu.sync_copy(r3, acc.at[pl.ds(15 * ZR + t * EC, EC)])

    plsc.subcore_barrier()

    def _body(i, carry):
        for j in range(NBUF):
            k = NBUF * i + j
            gwait(j)
            sstart(j)
            kn4 = k + NBUF

            @pl.when(kn4 < TCHUNKS)
            def _():
                idx_load(j, kn4)
            kn3 = k + 3
            j3 = (j + 3) % NBUF

            @pl.when(kn3 < TCHUNKS)
            def _():
                @pl.when(kn3 >= NBUF)
                def _():
                    swait(j3)
                idx_wait_adj(j3)
                gstart(j3)
        return carry
    lax.fori_loop(0, (TCHUNKS - 1) // NBUF, _body, 0)
    gwait(0)
    sstart(0)
    swait(1)
    swait(2)
    swait(3)
    swait(0)

    plsc.subcore_barrier()

    @pl.when(s < 15)
    def _():
        pltpu.sync_copy(acc.at[pl.ds(s * ZR, ZR)],
                        agg.at[c].at[pl.ds(s * ZR, ZR)])

    @pl.when(s == 15)
    def _():
        pltpu.sync_copy(acc.at[pl.ds(15 * ZR, N - 15 * ZR)],
                        agg.at[c].at[pl.ds(15 * ZR, N - 15 * ZR)])


def _make_edge_kernel():
    sem = pltpu.SemaphoreType.DMA
    return pl.kernel(
        _edge_body,
        out_type=jax.ShapeDtypeStruct((2, N, H), jnp.float32),
        mesh=_sc_mesh(),
        scratch_types=(
            [pltpu.VMEM_SHARED((N, H), jnp.float32)]
            + [pltpu.VMEM((EC,), jnp.int32) for _ in range(NBUF)]
            + [pltpu.VMEM((1, EC), jnp.int32) for _ in range(NBUF)]
            + [pltpu.VMEM((EC, H), jnp.float32) for _ in range(NBUF)]
            + [sem] * (3 * NBUF)
            + [pltpu.VMEM((1, EC), jnp.int32) for _ in range(NBUF)]

        ),
    )


def _mm0_body(deg_ref, x_ref, w_ref, h2_ref, g_ref, dis_ref):
    deg = deg_ref[0] + deg_ref[1] + 1.0
    dis = lax.rsqrt(deg)
    h2 = jnp.dot(x_ref[...], w_ref[...], preferred_element_type=jnp.float32)
    g = h2 * dis
    h2_ref[...] = h2.astype(jnp.bfloat16)
    g_ref[...] = jnp.stack([g[:, :H], g[:, H:]])
    dis_ref[...] = dis


def _mm_body(agg_ref, h2p_ref, dis_ref, b_ref, w_ref, h2_ref, g_ref):
    dis = dis_ref[...]
    agg = jnp.concatenate([agg_ref[0], agg_ref[1]], axis=-1)
    h2p = h2p_ref[...].astype(jnp.float32)
    hin = dis * agg + h2p * (dis * dis) + b_ref[...]
    hin = jnp.maximum(hin, 0.0)
    h2 = jnp.dot(hin, w_ref[...], preferred_element_type=jnp.float32)
    g = h2 * dis
    h2_ref[...] = h2.astype(jnp.bfloat16)
    g_ref[...] = jnp.stack([g[:, :H], g[:, H:]])


def _epi_body(agg_ref, h2p_ref, dis_ref, b_ref, out_ref):
    dis = dis_ref[...]
    agg = jnp.concatenate([agg_ref[0], agg_ref[1]], axis=-1)
    h2p = h2p_ref[...].astype(jnp.float32)
    out_ref[...] = dis * agg + h2p * (dis * dis) + b_ref[...]


def _row_spec(shape_tail):
    return pl.BlockSpec((BM,) + shape_tail,
                        lambda i: (i,) + (0,) * len(shape_tail))


def _pair_spec():
    return pl.BlockSpec((2, BM, H), lambda i: (0, i, 0))


def _make_mm0():
    return pl.pallas_call(
        _mm0_body,
        grid=(GRID,),
        in_specs=[
            pl.BlockSpec((2, BM, 1), lambda i: (0, i, 0)),
            _row_spec((D,)),
            pl.BlockSpec((D, D), lambda i: (0, 0)),
        ],
        out_specs=[_row_spec((D,)), _pair_spec(), _row_spec((1,))],
        out_shape=[jax.ShapeDtypeStruct((N, D), jnp.bfloat16),
                   jax.ShapeDtypeStruct((2, N, H), jnp.float32),
                   jax.ShapeDtypeStruct((N, 1), jnp.float32)],
    )


def _make_mm():
    return pl.pallas_call(
        _mm_body,
        grid=(GRID,),
        in_specs=[
            _pair_spec(), _row_spec((D,)), _row_spec((1,)),
            pl.BlockSpec((1, D), lambda i: (0, 0)),
            pl.BlockSpec((D, D), lambda i: (0, 0)),
        ],
        out_specs=[_row_spec((D,)), _pair_spec()],
        out_shape=[jax.ShapeDtypeStruct((N, D), jnp.bfloat16),
                   jax.ShapeDtypeStruct((2, N, H), jnp.float32)],
    )


def _make_epi():
    return pl.pallas_call(
        _epi_body,
        grid=(GRID,),
        in_specs=[
            _pair_spec(), _row_spec((D,)), _row_spec((1,)),
            pl.BlockSpec((1, D), lambda i: (0, 0)),
        ],
        out_specs=_row_spec((D,)),
        out_shape=jax.ShapeDtypeStruct((N, D), jnp.float32),
    )


def kernel(x, adj_t, W0, b0, W1, b1, W2, b2):
    src = adj_t[0]
    dst3 = adj_t[1].reshape(ECHUNKS, 1, EC)
    zeros1 = jnp.zeros((NPAD,), jnp.float32)
    ones1 = jnp.ones((DC,), jnp.float32)
    b0r = b0.reshape(1, D)
    b1r = b1.reshape(1, D)
    b2r = b2.reshape(1, D)

    deg_k = _make_deg_kernel()
    edge_k = _make_edge_kernel()
    mm0 = _make_mm0()
    mm = _make_mm()
    epi = _make_epi()

    deg1 = deg_k(adj_t[1].reshape(NC * NS, DTCHUNKS, DC), zeros1, ones1)
    h2, gp, dis = mm0(deg1.reshape(2, NPAD, 1)[:, :N], x, W0)
    aggp = edge_k(gp.reshape(2 * N, H), src, dst3)
    h2, gp = mm(aggp, h2, dis, b0r, W1)
    aggp = edge_k(gp.reshape(2 * N, H), src, dst3)
    h2, gp = mm(aggp, h2, dis, b1r, W2)
    aggp = edge_k(gp.reshape(2 * N, H), src, dst3)
    return epi(aggp, h2, dis, b2r)
